# baseline (speedup 1.0000x reference)
import functools

import jax
import jax.numpy as jnp
from jax import lax
from jax.experimental import pallas as pl
from jax.experimental.pallas import tpu as pltpu
from jax.experimental.pallas import tpu_sc as plsc

B, H, W, C, K = 8, 128, 128, 256, 21

_NUM_ROWS = B * K
_ROWS_PAD = 256
_R_PER_WORKER = 16
_ACTIVE_WORKERS = _ROWS_PAD // _R_PER_WORKER
_NC, _NS = 2, 16


@functools.partial(
    pl.kernel,
    out_type=jax.ShapeDtypeStruct((_ROWS_PAD, C), jnp.float32),
    mesh=plsc.VectorSubcoreMesh(core_axis_name="c", subcore_axis_name="s"),
    scratch_types=[
        pltpu.VMEM((2 * _ROWS_PAD,), jnp.int32),
        pltpu.VMEM((_R_PER_WORKER,), jnp.int32),
        pltpu.VMEM((_R_PER_WORKER, C), jnp.float32),
        pltpu.SemaphoreType.DMA,
    ],
)
def _gather_rows(hw_hbm, table_hbm, out_hbm, hw_v, idx_v, rows_v, sem):
    wid = lax.axis_index("s") * _NC + lax.axis_index("c")

    @pl.when(wid < _ACTIVE_WORKERS)
    def _():
        pltpu.sync_copy(hw_hbm, hw_v)
        base = wid * _R_PER_WORKER
        g = base + lax.iota(jnp.int32, 16)
        b = lax.div(g, jnp.int32(K))
        h = hw_v[pl.ds(base, _R_PER_WORKER)]
        w = hw_v[pl.ds(_ROWS_PAD + base, _R_PER_WORKER)]
        flat = b * (H * W) + h * W + w
        idx_v[...] = jnp.where(g < _NUM_ROWS, flat, 0)
        pltpu.async_copy(table_hbm.at[idx_v], rows_v, sem).wait()
        pltpu.sync_copy(rows_v, out_hbm.at[pl.ds(base, _R_PER_WORKER)])


def kernel(inputs, uv):
    table = inputs.reshape(B * H * W, C)
    uv32 = uv.astype(jnp.int32)
    pad = _ROWS_PAD - _NUM_ROWS
    h = jnp.pad(uv32[..., 0].reshape(_NUM_ROWS), (0, pad))
    w = jnp.pad(uv32[..., 1].reshape(_NUM_ROWS), (0, pad))
    hw = jnp.concatenate([h, w])
    out = _gather_rows(hw, table)
    return out[:_NUM_ROWS].reshape(B, K, C)
